# baseline (speedup 1.0000x reference)
import functools

import jax
import jax.numpy as jnp
from jax import lax
from jax.experimental import pallas as pl
from jax.experimental.pallas import tpu as pltpu
from jax.experimental.pallas import tpu_sc as plsc

N = 10000
D_IN = 128
D_HID = 128
D_OUT = 64
NC, NS = 2, 16
NW = NC * NS
CHUNK = 64
BLKC = 16
NBUF = 8
PREF = 6
N_ACC = 10240
ZROWS = N_ACC // NS
ROWS_INIT = N // NS
ROWS_OUT = N_ACC // NS

_mesh = plsc.VectorSubcoreMesh(core_axis_name="c", subcore_axis_name="s")
_params = pltpu.CompilerParams(use_tc_tiling_on_sc=False)


def _deg_body(nrw, dst_hbm, deg_out, dstv, ones, zbuf, degs, sem):
    c = lax.axis_index("c")
    s = lax.axis_index("s")
    w = c * NS + s
    cp = pltpu.async_copy(dst_hbm.at[pl.ds(w * nrw, nrw)], dstv, sem)

    zero16 = jnp.zeros((16,), jnp.float32)
    one16 = jnp.ones((16,), jnp.float32)

    def fbody(i, _):
        ones[pl.ds(i * 16, 16)] = one16
        return 0

    lax.fori_loop(0, CHUNK // 16, fbody, 0)

    def zbody(i, _):
        zbuf[pl.ds(i * 16, 16)] = zero16
        return 0

    lax.fori_loop(0, ZROWS // 16, zbody, 0)
    pltpu.sync_copy(zbuf, degs.at[pl.ds(s * ZROWS, ZROWS)])
    cp.wait()
    plsc.subcore_barrier()

    def chunk(j, _):
        pltpu.sync_copy(ones, degs.at[dstv.at[j]], add=True)
        return 0

    lax.fori_loop(0, nrw, chunk, 0)
    plsc.subcore_barrier()

    @pl.when(s == 0)
    def _():
        pltpu.sync_copy(degs, deg_out.at[c])


def _make_deg(nrw):
    return pl.kernel(
        functools.partial(_deg_body, nrw),
        out_type=jax.ShapeDtypeStruct((NC, N_ACC), jnp.float32),
        mesh=_mesh,
        compiler_params=_params,
        scratch_types=[
            pltpu.VMEM((nrw, CHUNK), jnp.int32),
            pltpu.VMEM((CHUNK,), jnp.float32),
            pltpu.VMEM((ZROWS,), jnp.float32),
            pltpu.VMEM_SHARED((N_ACC,), jnp.float32),
            pltpu.SemaphoreType.DMA,
        ],
    )


def _prop_body(nch, half, p_hbm, src_hbm, dst_hbm, g_out, srcv, dstv, rows,
               pstage, acc, isem, gsem, ssem, csem):
    c = lax.axis_index("c")
    s = lax.axis_index("s")
    nblk = nch // BLKC
    r0 = s * ROWS_INIT

    cp_p = pltpu.async_copy(p_hbm.at[c, pl.ds(r0, ROWS_INIT)],
                            pstage.at[pl.ds(r0, ROWS_INIT)], csem)
    cp_a = pltpu.async_copy(p_hbm.at[c, pl.ds(r0, ROWS_INIT)],
                            acc.at[pl.ds(r0, ROWS_INIT)], csem)

    def idx_dma(blk, ib):
        base = s * nch + blk * BLKC
        pltpu.async_copy(src_hbm.at[pl.ds(base, BLKC)], srcv.at[ib], isem)
        pltpu.async_copy(dst_hbm.at[pl.ds(base, BLKC)], dstv.at[ib], isem)

    def idx_wait(ib):
        pltpu.make_async_copy(src_hbm.at[pl.ds(0, BLKC)], srcv.at[ib],
                              isem).wait()
        pltpu.make_async_copy(dst_hbm.at[pl.ds(0, BLKC)], dstv.at[ib],
                              isem).wait()

    idx_dma(0, 0)
    cp_p.wait()
    cp_a.wait()
    plsc.subcore_barrier()

    def gather(ib, k, b):
        return pltpu.async_copy(pstage.at[srcv.at[ib, k]], rows.at[b], gsem)

    def gather_wait(b):
        pltpu.make_async_copy(pstage.at[srcv.at[0, 0]], rows.at[b],
                              gsem).wait()

    def scat_wait():
        pltpu.make_async_copy(rows.at[0], acc.at[dstv.at[0, 0]], ssem).wait()

    def blk_loop(blk, _):
        ib = blk & 1
        idx_wait(ib)

        @pl.when(blk + 1 < nblk)
        def _():
            idx_dma(blk + 1, 1 - ib)

        for k0 in range(PREF):
            gather(ib, k0, k0 % NBUF)
        for k in range(BLKC):
            b = k % NBUF
            if k == 0:
                @pl.when(blk >= 1)
                def _():
                    scat_wait()
            else:
                scat_wait()
            if k + PREF < BLKC:
                gather(ib, k + PREF, (k + PREF) % NBUF)
            gather_wait(b)
            pltpu.async_copy(rows.at[b], acc.at[dstv.at[ib, k]], ssem,
                             add=True)
        return 0

    lax.fori_loop(0, nblk, blk_loop, 0)
    scat_wait()
    plsc.subcore_barrier()
    pltpu.sync_copy(acc.at[pl.ds(s * ROWS_OUT, ROWS_OUT)],
                    g_out.at[c, pl.ds(s * ROWS_OUT, ROWS_OUT)])


def _make_prop(nch, half):
    return pl.kernel(
        functools.partial(_prop_body, nch, half),
        out_type=jax.ShapeDtypeStruct((NC, N_ACC, half), jnp.float32),
        mesh=_mesh,
        compiler_params=_params,
        scratch_types=[
            pltpu.VMEM((2, BLKC, CHUNK), jnp.int32),
            pltpu.VMEM((2, BLKC, CHUNK), jnp.int32),
            pltpu.VMEM((NBUF, CHUNK, half), jnp.float32),
            pltpu.VMEM_SHARED((N_ACC, half), jnp.float32),
            pltpu.VMEM_SHARED((N_ACC, half), jnp.float32),
            pltpu.SemaphoreType.DMA,
            pltpu.SemaphoreType.DMA,
            pltpu.SemaphoreType.DMA,
            pltpu.SemaphoreType.DMA,
        ],
    )


_BLK = 2000
_GRID = N // _BLK


def _split_cols(p_ref, p):
    h2 = p.shape[1] // 2
    p_ref[0, ...] = p[:, :h2]
    p_ref[1, ...] = p[:, h2:]


def _prep_body(deg_ref, x_ref, w_ref, s_ref, p_ref):
    sv = lax.rsqrt(deg_ref[0, ...] + deg_ref[1, ...] + 1.0)
    h = jnp.dot(x_ref[...], w_ref[...], preferred_element_type=jnp.float32)
    s_ref[...] = sv
    _split_cols(p_ref, h * sv)


def _layer_body(g_ref, s_ref, b_ref, w_ref, p_ref):
    g = jnp.concatenate([g_ref[0, ...], g_ref[1, ...]], axis=1)
    h = jnp.maximum(g * s_ref[...] + b_ref[...], 0.0)
    _split_cols(p_ref, jnp.dot(h, w_ref[...],
                               preferred_element_type=jnp.float32) * s_ref[...])


def _final_body(g_ref, s_ref, b_ref, o_ref):
    v = jnp.concatenate([g_ref[0, ...], g_ref[1, ...]], axis=1)
    v = v * s_ref[...] + b_ref[...]
    m = jnp.max(v, axis=1, keepdims=True)
    z = v - m
    o_ref[...] = z - jnp.log(jnp.sum(jnp.exp(z), axis=1, keepdims=True))


def _row_spec(d):
    return pl.BlockSpec((_BLK, d), lambda i: (i, 0))


def _half_spec(d):
    return pl.BlockSpec((2, _BLK, d // 2), lambda i: (0, i, 0))


def _rep_spec(r, d):
    return pl.BlockSpec((r, d), lambda i: (0, 0))


def _prep(deg, x, w):
    return pl.pallas_call(
        _prep_body,
        grid=(_GRID,),
        in_specs=[pl.BlockSpec((2, _BLK, 1), lambda i: (0, i, 0)),
                  _row_spec(D_IN), _rep_spec(D_IN, D_HID)],
        out_specs=[_row_spec(1), _half_spec(D_HID)],
        out_shape=[jax.ShapeDtypeStruct((N, 1), jnp.float32),
                   jax.ShapeDtypeStruct((2, N, D_HID // 2), jnp.float32)],
    )(deg, x, w)


def _layer(g, sv, b, w, d_in, d_out):
    return pl.pallas_call(
        _layer_body,
        grid=(_GRID,),
        in_specs=[_half_spec(d_in), _row_spec(1), _rep_spec(1, d_in),
                  _rep_spec(d_in, d_out)],
        out_specs=_half_spec(d_out),
        out_shape=jax.ShapeDtypeStruct((2, N, d_out // 2), jnp.float32),
    )(g, sv, b.reshape(1, d_in), w)


def _final(g, sv, b):
    return pl.pallas_call(
        _final_body,
        grid=(_GRID,),
        in_specs=[_half_spec(D_OUT), _row_spec(1), _rep_spec(1, D_OUT)],
        out_specs=_row_spec(D_OUT),
        out_shape=jax.ShapeDtypeStruct((N, D_OUT), jnp.float32),
    )(g, sv, b.reshape(1, D_OUT))


def kernel(x, edge_index, W1, b1, W2, b2, W3, b3):
    e = edge_index.shape[1]
    nch = -(-e // (NS * CHUNK))
    nch = -(-nch // BLKC) * BLKC
    per_t = nch * CHUNK
    ei = edge_index.astype(jnp.int32)
    e_t = e // NS
    pad_t = per_t - e_t
    pad_dst = N + (jnp.arange(pad_t, dtype=jnp.int32) % (N_ACC - N))
    src2d = jnp.concatenate(
        [ei[0].reshape(NS, e_t), jnp.zeros((NS, pad_t), jnp.int32)],
        axis=1).reshape(NS * nch, CHUNK)
    dst2d = jnp.concatenate(
        [ei[1].reshape(NS, e_t), jnp.broadcast_to(pad_dst, (NS, pad_t))],
        axis=1).reshape(NS * nch, CHUNK)

    deg2 = _make_deg(NS * nch // NW)(dst2d)

    sv, p1 = _prep(deg2.reshape(NC, N_ACC, 1), x, W1)

    g = _make_prop(nch, D_HID // 2)(p1, src2d, dst2d)
    p2 = _layer(g, sv, b1, W2, D_HID, D_HID)

    g = _make_prop(nch, D_HID // 2)(p2, src2d, dst2d)
    p3 = _layer(g, sv, b2, W3, D_HID, D_OUT)

    g = _make_prop(nch, D_OUT // 2)(p3, src2d, dst2d)
    return _final(g, sv, b3)


# speedup vs baseline: 1.0633x; 1.0633x over previous
import functools

import jax
import jax.numpy as jnp
from jax import lax
from jax.experimental import pallas as pl
from jax.experimental.pallas import tpu as pltpu
from jax.experimental.pallas import tpu_sc as plsc

N = 10000
D_IN = 128
D_HID = 128
D_OUT = 64
NC, NS = 2, 16
NW = NC * NS
CHUNK = 128
BLKC = 16
NBUF = 4
PREF = 3
N_ACC = 10240
ZROWS = N_ACC // NS
ROWS_INIT = N // NS
ROWS_OUT = N_ACC // NS

_mesh = plsc.VectorSubcoreMesh(core_axis_name="c", subcore_axis_name="s")
_params = pltpu.CompilerParams(use_tc_tiling_on_sc=False)


def _deg_body(nrw, dst_hbm, deg_out, dstv, ones, zbuf, degs, sem):
    c = lax.axis_index("c")
    s = lax.axis_index("s")
    w = c * NS + s
    cp = pltpu.async_copy(dst_hbm.at[pl.ds(w * nrw, nrw)], dstv, sem)

    zero16 = jnp.zeros((16,), jnp.float32)
    one16 = jnp.ones((16,), jnp.float32)

    def fbody(i, _):
        ones[pl.ds(i * 16, 16)] = one16
        return 0

    lax.fori_loop(0, CHUNK // 16, fbody, 0)

    def zbody(i, _):
        zbuf[pl.ds(i * 16, 16)] = zero16
        return 0

    lax.fori_loop(0, ZROWS // 16, zbody, 0)
    pltpu.sync_copy(zbuf, degs.at[pl.ds(s * ZROWS, ZROWS)])
    cp.wait()
    plsc.subcore_barrier()

    def chunk(j, _):
        pltpu.sync_copy(ones, degs.at[dstv.at[j]], add=True)
        return 0

    lax.fori_loop(0, nrw, chunk, 0)
    plsc.subcore_barrier()

    @pl.when(s == 0)
    def _():
        pltpu.sync_copy(degs, deg_out.at[c])


def _make_deg(nrw):
    return pl.kernel(
        functools.partial(_deg_body, nrw),
        out_type=jax.ShapeDtypeStruct((NC, N_ACC), jnp.float32),
        mesh=_mesh,
        compiler_params=_params,
        scratch_types=[
            pltpu.VMEM((nrw, CHUNK), jnp.int32),
            pltpu.VMEM((CHUNK,), jnp.float32),
            pltpu.VMEM((ZROWS,), jnp.float32),
            pltpu.VMEM_SHARED((N_ACC,), jnp.float32),
            pltpu.SemaphoreType.DMA,
        ],
    )


def _prop_body(nch, half, p_hbm, src_hbm, dst_hbm, g_out, srcv, dstv, rows,
               pstage, acc, isem, gsem, ssem, csem):
    c = lax.axis_index("c")
    s = lax.axis_index("s")
    nblk = nch // BLKC
    r0 = s * ROWS_INIT

    cp_p = pltpu.async_copy(p_hbm.at[c, pl.ds(r0, ROWS_INIT)],
                            pstage.at[pl.ds(r0, ROWS_INIT)], csem)
    cp_a = pltpu.async_copy(p_hbm.at[c, pl.ds(r0, ROWS_INIT)],
                            acc.at[pl.ds(r0, ROWS_INIT)], csem)

    def idx_dma(blk, ib):
        base = s * nch + blk * BLKC
        pltpu.async_copy(src_hbm.at[pl.ds(base, BLKC)], srcv.at[ib], isem)
        pltpu.async_copy(dst_hbm.at[pl.ds(base, BLKC)], dstv.at[ib], isem)

    def idx_wait(ib):
        pltpu.make_async_copy(src_hbm.at[pl.ds(0, BLKC)], srcv.at[ib],
                              isem).wait()
        pltpu.make_async_copy(dst_hbm.at[pl.ds(0, BLKC)], dstv.at[ib],
                              isem).wait()

    idx_dma(0, 0)
    cp_p.wait()
    cp_a.wait()
    plsc.subcore_barrier()

    def gather(ib, k, b):
        return pltpu.async_copy(pstage.at[srcv.at[ib, k]], rows.at[b], gsem)

    def gather_wait(b):
        pltpu.make_async_copy(pstage.at[srcv.at[0, 0]], rows.at[b],
                              gsem).wait()

    def scat_wait():
        pltpu.make_async_copy(rows.at[0], acc.at[dstv.at[0, 0]], ssem).wait()

    def blk_loop(blk, _):
        ib = blk & 1
        idx_wait(ib)

        @pl.when(blk + 1 < nblk)
        def _():
            idx_dma(blk + 1, 1 - ib)

        for k0 in range(PREF):
            gather(ib, k0, k0 % NBUF)
        for k in range(BLKC):
            b = k % NBUF
            if k == 0:
                @pl.when(blk >= 1)
                def _():
                    scat_wait()
            else:
                scat_wait()
            if k + PREF < BLKC:
                gather(ib, k + PREF, (k + PREF) % NBUF)
            gather_wait(b)
            pltpu.async_copy(rows.at[b], acc.at[dstv.at[ib, k]], ssem,
                             add=True)
        return 0

    lax.fori_loop(0, nblk, blk_loop, 0)
    scat_wait()
    plsc.subcore_barrier()
    pltpu.sync_copy(acc.at[pl.ds(s * ROWS_OUT, ROWS_OUT)],
                    g_out.at[c, pl.ds(s * ROWS_OUT, ROWS_OUT)])


def _make_prop(nch, half):
    return pl.kernel(
        functools.partial(_prop_body, nch, half),
        out_type=jax.ShapeDtypeStruct((NC, N_ACC, half), jnp.float32),
        mesh=_mesh,
        compiler_params=_params,
        scratch_types=[
            pltpu.VMEM((2, BLKC, CHUNK), jnp.int32),
            pltpu.VMEM((2, BLKC, CHUNK), jnp.int32),
            pltpu.VMEM((NBUF, CHUNK, half), jnp.float32),
            pltpu.VMEM_SHARED((N_ACC, half), jnp.float32),
            pltpu.VMEM_SHARED((N_ACC, half), jnp.float32),
            pltpu.SemaphoreType.DMA,
            pltpu.SemaphoreType.DMA,
            pltpu.SemaphoreType.DMA,
            pltpu.SemaphoreType.DMA,
        ],
    )


_BLK = 2000
_GRID = N // _BLK


def _split_cols(p_ref, p):
    h2 = p.shape[1] // 2
    p_ref[0, ...] = p[:, :h2]
    p_ref[1, ...] = p[:, h2:]


def _prep_body(deg_ref, x_ref, w_ref, s_ref, p_ref):
    sv = lax.rsqrt(deg_ref[0, ...] + deg_ref[1, ...] + 1.0)
    h = jnp.dot(x_ref[...], w_ref[...], preferred_element_type=jnp.float32)
    s_ref[...] = sv
    _split_cols(p_ref, h * sv)


def _layer_body(g_ref, s_ref, b_ref, w_ref, p_ref):
    g = jnp.concatenate([g_ref[0, ...], g_ref[1, ...]], axis=1)
    h = jnp.maximum(g * s_ref[...] + b_ref[...], 0.0)
    _split_cols(p_ref, jnp.dot(h, w_ref[...],
                               preferred_element_type=jnp.float32) * s_ref[...])


def _final_body(g_ref, s_ref, b_ref, o_ref):
    v = jnp.concatenate([g_ref[0, ...], g_ref[1, ...]], axis=1)
    v = v * s_ref[...] + b_ref[...]
    m = jnp.max(v, axis=1, keepdims=True)
    z = v - m
    o_ref[...] = z - jnp.log(jnp.sum(jnp.exp(z), axis=1, keepdims=True))


def _row_spec(d):
    return pl.BlockSpec((_BLK, d), lambda i: (i, 0))


def _half_spec(d):
    return pl.BlockSpec((2, _BLK, d // 2), lambda i: (0, i, 0))


def _rep_spec(r, d):
    return pl.BlockSpec((r, d), lambda i: (0, 0))


def _prep(deg, x, w):
    return pl.pallas_call(
        _prep_body,
        grid=(_GRID,),
        in_specs=[pl.BlockSpec((2, _BLK, 1), lambda i: (0, i, 0)),
                  _row_spec(D_IN), _rep_spec(D_IN, D_HID)],
        out_specs=[_row_spec(1), _half_spec(D_HID)],
        out_shape=[jax.ShapeDtypeStruct((N, 1), jnp.float32),
                   jax.ShapeDtypeStruct((2, N, D_HID // 2), jnp.float32)],
    )(deg, x, w)


def _layer(g, sv, b, w, d_in, d_out):
    return pl.pallas_call(
        _layer_body,
        grid=(_GRID,),
        in_specs=[_half_spec(d_in), _row_spec(1), _rep_spec(1, d_in),
                  _rep_spec(d_in, d_out)],
        out_specs=_half_spec(d_out),
        out_shape=jax.ShapeDtypeStruct((2, N, d_out // 2), jnp.float32),
    )(g, sv, b.reshape(1, d_in), w)


def _final(g, sv, b):
    return pl.pallas_call(
        _final_body,
        grid=(_GRID,),
        in_specs=[_half_spec(D_OUT), _row_spec(1), _rep_spec(1, D_OUT)],
        out_specs=_row_spec(D_OUT),
        out_shape=jax.ShapeDtypeStruct((N, D_OUT), jnp.float32),
    )(g, sv, b.reshape(1, D_OUT))


def kernel(x, edge_index, W1, b1, W2, b2, W3, b3):
    e = edge_index.shape[1]
    nch = -(-e // (NS * CHUNK))
    nch = -(-nch // BLKC) * BLKC
    per_t = nch * CHUNK
    ei = edge_index.astype(jnp.int32)
    e_t = e // NS
    pad_t = per_t - e_t
    pad_dst = N + (jnp.arange(pad_t, dtype=jnp.int32) % (N_ACC - N))
    src2d = jnp.concatenate(
        [ei[0].reshape(NS, e_t), jnp.zeros((NS, pad_t), jnp.int32)],
        axis=1).reshape(NS * nch, CHUNK)
    dst2d = jnp.concatenate(
        [ei[1].reshape(NS, e_t), jnp.broadcast_to(pad_dst, (NS, pad_t))],
        axis=1).reshape(NS * nch, CHUNK)

    deg2 = _make_deg(NS * nch // NW)(dst2d)

    sv, p1 = _prep(deg2.reshape(NC, N_ACC, 1), x, W1)

    g = _make_prop(nch, D_HID // 2)(p1, src2d, dst2d)
    p2 = _layer(g, sv, b1, W2, D_HID, D_HID)

    g = _make_prop(nch, D_HID // 2)(p2, src2d, dst2d)
    p3 = _layer(g, sv, b2, W3, D_HID, D_OUT)

    g = _make_prop(nch, D_OUT // 2)(p3, src2d, dst2d)
    return _final(g, sv, b3)


# speedup vs baseline: 1.0809x; 1.0166x over previous
import functools

import jax
import jax.numpy as jnp
from jax import lax
from jax.experimental import pallas as pl
from jax.experimental.pallas import tpu as pltpu
from jax.experimental.pallas import tpu_sc as plsc

N = 10000
D_IN = 128
D_HID = 128
D_OUT = 64
NC, NS = 2, 16
NW = NC * NS
CHUNK = 128
BLKC = 16
NBUF = 4
PREF = 3
N_ACC = 10240
ZROWS = N_ACC // NS
ROWS_INIT = N // NS
ROWS_OUT = N_ACC // NS

_mesh = plsc.VectorSubcoreMesh(core_axis_name="c", subcore_axis_name="s")
_params = pltpu.CompilerParams(use_tc_tiling_on_sc=False)


def _deg_body(nrw, dst_hbm, deg_out, dstv, ones, zbuf, degs, sem):
    c = lax.axis_index("c")
    s = lax.axis_index("s")
    w = c * NS + s
    cp = pltpu.async_copy(dst_hbm.at[pl.ds(w * nrw, nrw)], dstv, sem)

    zero16 = jnp.zeros((16,), jnp.float32)
    one16 = jnp.ones((16,), jnp.float32)

    def fbody(i, _):
        ones[pl.ds(i * 16, 16)] = one16
        return 0

    lax.fori_loop(0, CHUNK // 16, fbody, 0)

    def zbody(i, _):
        zbuf[pl.ds(i * 16, 16)] = zero16
        return 0

    lax.fori_loop(0, ZROWS // 16, zbody, 0)
    pltpu.sync_copy(zbuf, degs.at[pl.ds(s * ZROWS, ZROWS)])
    cp.wait()
    plsc.subcore_barrier()

    def chunk(j, _):
        pltpu.sync_copy(ones, degs.at[dstv.at[j]], add=True)
        return 0

    lax.fori_loop(0, nrw, chunk, 0)
    plsc.subcore_barrier()

    @pl.when(s == 0)
    def _():
        pltpu.sync_copy(degs, deg_out.at[c])


def _make_deg(nrw):
    return pl.kernel(
        functools.partial(_deg_body, nrw),
        out_type=jax.ShapeDtypeStruct((NC, N_ACC), jnp.float32),
        mesh=_mesh,
        compiler_params=_params,
        scratch_types=[
            pltpu.VMEM((nrw, CHUNK), jnp.int32),
            pltpu.VMEM((CHUNK,), jnp.float32),
            pltpu.VMEM((ZROWS,), jnp.float32),
            pltpu.VMEM_SHARED((N_ACC,), jnp.float32),
            pltpu.SemaphoreType.DMA,
        ],
    )


def _prop_body(nch, half, p_hbm, src_hbm, dst_hbm, g_out, srcv, dstv, rows,
               pstage, acc, isem, gsem, ssem, csem):
    c = lax.axis_index("c")
    s = lax.axis_index("s")
    nblk = nch // BLKC
    r0 = s * ROWS_INIT

    cp_p = pltpu.async_copy(p_hbm.at[c, pl.ds(r0, ROWS_INIT)],
                            pstage.at[pl.ds(r0, ROWS_INIT)], csem)
    cp_a = pltpu.async_copy(p_hbm.at[c, pl.ds(r0, ROWS_INIT)],
                            acc.at[pl.ds(r0, ROWS_INIT)], csem)

    def idx_dma(blk, ib):
        base = s * nch + blk * BLKC
        pltpu.async_copy(src_hbm.at[pl.ds(base, BLKC)], srcv.at[ib], isem)
        pltpu.async_copy(dst_hbm.at[pl.ds(base, BLKC)], dstv.at[ib], isem)

    def idx_wait(ib):
        pltpu.make_async_copy(src_hbm.at[pl.ds(0, BLKC)], srcv.at[ib],
                              isem).wait()
        pltpu.make_async_copy(dst_hbm.at[pl.ds(0, BLKC)], dstv.at[ib],
                              isem).wait()

    idx_dma(0, 0)
    cp_p.wait()
    cp_a.wait()
    plsc.subcore_barrier()

    def gather(ib, k, b):
        return pltpu.async_copy(pstage.at[srcv.at[ib, k]], rows.at[b], gsem)

    def gather_wait(b):
        pltpu.make_async_copy(pstage.at[srcv.at[0, 0]], rows.at[b],
                              gsem).wait()

    def scat_wait():
        pltpu.make_async_copy(rows.at[0], acc.at[dstv.at[0, 0]], ssem).wait()

    def blk_loop(blk, _):
        ib = blk & 1
        idx_wait(ib)

        @pl.when(blk + 1 < nblk)
        def _():
            idx_dma(blk + 1, 1 - ib)

        for k0 in range(PREF):
            gather(ib, k0, k0 % NBUF)
        for k in range(BLKC):
            b = k % NBUF
            if k == 0:
                @pl.when(blk >= 1)
                def _():
                    scat_wait()
            else:
                scat_wait()
            if k + PREF < BLKC:
                gather(ib, k + PREF, (k + PREF) % NBUF)
            gather_wait(b)
            pltpu.async_copy(rows.at[b], acc.at[dstv.at[ib, k]], ssem,
                             add=True)
        return 0

    lax.fori_loop(0, nblk, blk_loop, 0)
    scat_wait()
    plsc.subcore_barrier()
    pltpu.sync_copy(acc.at[pl.ds(s * ROWS_OUT, ROWS_OUT)],
                    g_out.at[c, pl.ds(s * ROWS_OUT, ROWS_OUT)])


def _make_prop(nch, half):
    return pl.kernel(
        functools.partial(_prop_body, nch, half),
        out_type=jax.ShapeDtypeStruct((NC, N_ACC, half), jnp.float32),
        mesh=_mesh,
        compiler_params=_params,
        scratch_types=[
            pltpu.VMEM((2, BLKC, CHUNK), jnp.int32),
            pltpu.VMEM((2, BLKC, CHUNK), jnp.int32),
            pltpu.VMEM((NBUF, CHUNK, half), jnp.float32),
            pltpu.VMEM_SHARED((N_ACC, half), jnp.float32),
            pltpu.VMEM_SHARED((N_ACC, half), jnp.float32),
            pltpu.SemaphoreType.DMA,
            pltpu.SemaphoreType.DMA,
            pltpu.SemaphoreType.DMA,
            pltpu.SemaphoreType.DMA,
        ],
    )


_BLK = 5000
_GRID = N // _BLK


def _split_cols(p_ref, p):
    h2 = p.shape[1] // 2
    p_ref[0, ...] = p[:, :h2]
    p_ref[1, ...] = p[:, h2:]


def _prep_body(deg_ref, x_ref, w_ref, s_ref, p_ref):
    sv = lax.rsqrt(deg_ref[0, ...] + deg_ref[1, ...] + 1.0)
    h = jnp.dot(x_ref[...], w_ref[...], preferred_element_type=jnp.float32)
    s_ref[...] = sv
    _split_cols(p_ref, h * sv)


def _layer_body(g_ref, s_ref, b_ref, w_ref, p_ref):
    g = jnp.concatenate([g_ref[0, ...], g_ref[1, ...]], axis=1)
    h = jnp.maximum(g * s_ref[...] + b_ref[...], 0.0)
    _split_cols(p_ref, jnp.dot(h, w_ref[...],
                               preferred_element_type=jnp.float32) * s_ref[...])


def _final_body(g_ref, s_ref, b_ref, o_ref):
    v = jnp.concatenate([g_ref[0, ...], g_ref[1, ...]], axis=1)
    v = v * s_ref[...] + b_ref[...]
    m = jnp.max(v, axis=1, keepdims=True)
    z = v - m
    o_ref[...] = z - jnp.log(jnp.sum(jnp.exp(z), axis=1, keepdims=True))


def _row_spec(d):
    return pl.BlockSpec((_BLK, d), lambda i: (i, 0))


def _half_spec(d):
    return pl.BlockSpec((2, _BLK, d // 2), lambda i: (0, i, 0))


def _rep_spec(r, d):
    return pl.BlockSpec((r, d), lambda i: (0, 0))


def _prep(deg, x, w):
    return pl.pallas_call(
        _prep_body,
        grid=(_GRID,),
        in_specs=[pl.BlockSpec((2, _BLK, 1), lambda i: (0, i, 0)),
                  _row_spec(D_IN), _rep_spec(D_IN, D_HID)],
        out_specs=[_row_spec(1), _half_spec(D_HID)],
        out_shape=[jax.ShapeDtypeStruct((N, 1), jnp.float32),
                   jax.ShapeDtypeStruct((2, N, D_HID // 2), jnp.float32)],
    )(deg, x, w)


def _layer(g, sv, b, w, d_in, d_out):
    return pl.pallas_call(
        _layer_body,
        grid=(_GRID,),
        in_specs=[_half_spec(d_in), _row_spec(1), _rep_spec(1, d_in),
                  _rep_spec(d_in, d_out)],
        out_specs=_half_spec(d_out),
        out_shape=jax.ShapeDtypeStruct((2, N, d_out // 2), jnp.float32),
    )(g, sv, b.reshape(1, d_in), w)


def _final(g, sv, b):
    return pl.pallas_call(
        _final_body,
        grid=(_GRID,),
        in_specs=[_half_spec(D_OUT), _row_spec(1), _rep_spec(1, D_OUT)],
        out_specs=_row_spec(D_OUT),
        out_shape=jax.ShapeDtypeStruct((N, D_OUT), jnp.float32),
    )(g, sv, b.reshape(1, D_OUT))


def kernel(x, edge_index, W1, b1, W2, b2, W3, b3):
    e = edge_index.shape[1]
    nch = -(-e // (NS * CHUNK))
    nch = -(-nch // BLKC) * BLKC
    per_t = nch * CHUNK
    ei = edge_index.astype(jnp.int32)
    e_t = e // NS
    pad_t = per_t - e_t
    pad_dst = N + (jnp.arange(pad_t, dtype=jnp.int32) % (N_ACC - N))
    src2d = jnp.concatenate(
        [ei[0].reshape(NS, e_t), jnp.zeros((NS, pad_t), jnp.int32)],
        axis=1).reshape(NS * nch, CHUNK)
    dst2d = jnp.concatenate(
        [ei[1].reshape(NS, e_t), jnp.broadcast_to(pad_dst, (NS, pad_t))],
        axis=1).reshape(NS * nch, CHUNK)

    deg2 = _make_deg(NS * nch // NW)(dst2d)

    sv, p1 = _prep(deg2.reshape(NC, N_ACC, 1), x, W1)

    g = _make_prop(nch, D_HID // 2)(p1, src2d, dst2d)
    p2 = _layer(g, sv, b1, W2, D_HID, D_HID)

    g = _make_prop(nch, D_HID // 2)(p2, src2d, dst2d)
    p3 = _layer(g, sv, b2, W3, D_HID, D_OUT)

    g = _make_prop(nch, D_OUT // 2)(p3, src2d, dst2d)
    return _final(g, sv, b3)
